# baseline (speedup 1.0000x reference)
import functools

import jax
import jax.numpy as jnp
from jax import lax
from jax.experimental import pallas as pl
from jax.experimental.pallas import tpu as pltpu
from jax.experimental.pallas import tpu_sc as plsc

N = 10000
E = 320000
D_IN = 128
D_H = 128
D_OUT = 64
DW = 64
NUM_LABEL_LAYERS = 10

NC = 2
NS = 16
NW = NC * NS
EPW = E // NW
K = 72
NCHUNK = EPW // K
KTAIL = EPW - NCHUNK * K
RPT = 624
RTAIL = N - NS * RPT
DEGW = 4

_f32 = jnp.float32


def _worker_id():
    return lax.axis_index("s") * NC + lax.axis_index("c")


def _make_agg(D):
    mesh = plsc.VectorSubcoreMesh(core_axis_name="c", subcore_axis_name="s", num_cores=NC, num_subcores=NS)

    @functools.partial(
        pl.kernel,
        out_type=jax.ShapeDtypeStruct((NC, N, D), _f32),
        mesh=mesh,
        compiler_params=pltpu.CompilerParams(use_tc_tiling_on_sc=False),
        scratch_types=[
            pltpu.VMEM((NCHUNK, K), jnp.int32),
            pltpu.VMEM((NCHUNK, K), jnp.int32),
            pltpu.VMEM((1, KTAIL), jnp.int32),
            pltpu.VMEM((1, KTAIL), jnp.int32),
            pltpu.VMEM((3, K, D), _f32),
            pltpu.VMEM_SHARED((N, D), _f32),
            [pltpu.SemaphoreType.DMA] * 3,
            [pltpu.SemaphoreType.DMA] * 3,
        ],
    )
    def agg(xws_hbm, row_hbm, col_hbm, rowt_hbm, colt_hbm, out_hbm,
            row_v, col_v, rowt_v, colt_v, buf_v, acc_s, gsem, tsem):
        c = lax.axis_index("c")
        s = lax.axis_index("s")
        wid = s * NC + c
        pltpu.sync_copy(row_hbm.at[wid], row_v)
        pltpu.sync_copy(col_hbm.at[wid], col_v)
        pltpu.sync_copy(rowt_hbm.at[wid], rowt_v)
        pltpu.sync_copy(colt_hbm.at[wid], colt_v)
        base = pl.multiple_of(s * RPT, 8)
        pltpu.sync_copy(xws_hbm.at[pl.ds(base, RPT)],
                        acc_s.at[pl.ds(base, RPT)])

        @pl.when(s == 0)
        def _():
            pltpu.sync_copy(xws_hbm.at[pl.ds(NS * RPT, RTAIL)],
                            acc_s.at[pl.ds(NS * RPT, RTAIL)])

        plsc.subcore_barrier()

        def gather(j, slot):
            return pltpu.make_async_copy(
                xws_hbm.at[row_v.at[j]], buf_v.at[slot], gsem[slot])

        def scatter(j, slot):
            return pltpu.make_async_copy(
                buf_v.at[slot], acc_s.at[col_v.at[j]], tsem[slot])

        gather(0, 0).start()
        gather(1, 1).start()

        def tri(q, carry):
            for k in range(3):
                j = 3 * q + k
                gather(j, k).wait()
                scatter(j, k).start(add=True)
                more = j + 2 < NCHUNK

                @pl.when(more & (j >= 1))
                def _():
                    scatter(j - 1, (k + 2) % 3).wait()
                    gather(j + 2, (k + 2) % 3).start()

                @pl.when(more & (j < 1))
                def _():
                    gather(j + 2, (k + 2) % 3).start()
            return carry

        lax.fori_loop(0, NCHUNK // 3, tri, 0)
        for j in range(3 * (NCHUNK // 3), NCHUNK):
            gather(j, j % 3).wait()
            scatter(j, j % 3).start(add=True)
        for j in range(max(0, NCHUNK - 3), NCHUNK):
            scatter(j, j % 3).wait()
        pltpu.sync_copy(xws_hbm.at[rowt_v.at[0]], buf_v.at[0, pl.ds(0, KTAIL)])
        pltpu.sync_copy(buf_v.at[0, pl.ds(0, KTAIL)],
                        acc_s.at[colt_v.at[0]], add=True)
        plsc.subcore_barrier()
        pltpu.sync_copy(acc_s.at[pl.ds(base, RPT)],
                        out_hbm.at[c, pl.ds(base, RPT)])

        @pl.when(s == 0)
        def _():
            pltpu.sync_copy(acc_s.at[pl.ds(NS * RPT, RTAIL)],
                            out_hbm.at[c, pl.ds(NS * RPT, RTAIL)])

    return agg


_get_agg = functools.lru_cache(maxsize=None)(_make_agg)


def _make_deg():
    mesh = plsc.VectorSubcoreMesh(core_axis_name="c", subcore_axis_name="s", num_cores=NC, num_subcores=NS)

    @functools.partial(
        pl.kernel,
        out_type=jax.ShapeDtypeStruct((NC, N, DEGW), _f32),
        mesh=mesh,
        compiler_params=pltpu.CompilerParams(use_tc_tiling_on_sc=False),
        scratch_types=[
            pltpu.VMEM((NCHUNK, K), jnp.int32),
            pltpu.VMEM((1, KTAIL), jnp.int32),
            pltpu.VMEM((K, DEGW), _f32),
            pltpu.VMEM_SHARED((N, DEGW), _f32),
            pltpu.SemaphoreType.DMA,
        ],
    )
    def deg(ones_hbm, col_hbm, colt_hbm, out_hbm, col_v, colt_v, ones_v,
            acc_s, sem):
        c = lax.axis_index("c")
        s = lax.axis_index("s")
        wid = s * NC + c
        pltpu.sync_copy(col_hbm.at[wid], col_v)
        pltpu.sync_copy(colt_hbm.at[wid], colt_v)
        pltpu.sync_copy(ones_hbm.at[pl.ds(0, K)], ones_v)
        base = pl.multiple_of(s * RPT, 8)
        pltpu.sync_copy(ones_hbm.at[pl.ds(base, RPT)],
                        acc_s.at[pl.ds(base, RPT)])

        @pl.when(s == 0)
        def _():
            pltpu.sync_copy(ones_hbm.at[pl.ds(NS * RPT, RTAIL)],
                            acc_s.at[pl.ds(NS * RPT, RTAIL)])

        plsc.subcore_barrier()

        def scat(j):
            return pltpu.make_async_copy(ones_v, acc_s.at[col_v.at[j]], sem)

        def start_body(j, carry):
            scat(j).start(add=True)
            return carry

        lax.fori_loop(0, NCHUNK, start_body, 0)
        tail = pltpu.make_async_copy(ones_v.at[pl.ds(0, KTAIL)],
                                     acc_s.at[colt_v.at[0]], sem)
        tail.start(add=True)

        def wait_body(j, carry):
            scat(0).wait()
            return carry

        lax.fori_loop(0, NCHUNK, wait_body, 0)
        tail.wait()
        plsc.subcore_barrier()
        pltpu.sync_copy(acc_s.at[pl.ds(base, RPT)],
                        out_hbm.at[c, pl.ds(base, RPT)])

        @pl.when(s == 0)
        def _():
            pltpu.sync_copy(acc_s.at[pl.ds(NS * RPT, RTAIL)],
                            out_hbm.at[c, pl.ds(NS * RPT, RTAIL)])

    return deg


_get_deg = functools.lru_cache(maxsize=None)(_make_deg)


R = 5000
G = N // R
DISW = 8


def _rows(i):
    return (i, 0)


def _rows3(i):
    return (0, i, 0)


def _full(i):
    return (0, 0)


def _stage_a_body(x_ref, w_ref, dp_ref, dis_ref, xws_ref):
    dp = dp_ref[...]
    deg = (dp[0] + dp[1])[:, 0:1] - 1.0
    dis = lax.rsqrt(deg)
    dis_ref[...] = jnp.broadcast_to(dis, (R, DISW))
    xws_ref[...] = dis * jnp.dot(x_ref[...], w_ref[...],
                                 preferred_element_type=_f32)


_stage_a = pl.pallas_call(
    _stage_a_body,
    grid=(G,),
    in_specs=[
        pl.BlockSpec((R, D_IN), _rows),
        pl.BlockSpec((D_IN, D_H), _full),
        pl.BlockSpec((NC, R, DEGW), _rows3),
    ],
    out_specs=[pl.BlockSpec((R, DISW), _rows), pl.BlockSpec((R, D_H), _rows)],
    out_shape=[jax.ShapeDtypeStruct((N, DISW), _f32),
               jax.ShapeDtypeStruct((N, D_H), _f32)],
)


def _stage_b1_body(q_ref, xws_ref, b_ref, dis_ref, w_ref, out_ref):
    q = q_ref[...]
    dis = dis_ref[...][:, 0:1]
    h = dis * (q[0] + q[1] - xws_ref[...]) + b_ref[...]
    h = jnp.maximum(h, 0.0)
    out_ref[...] = dis * jnp.dot(h, w_ref[...], preferred_element_type=_f32)


_stage_b1 = pl.pallas_call(
    _stage_b1_body,
    grid=(G,),
    in_specs=[
        pl.BlockSpec((NC, R, D_H), _rows3),
        pl.BlockSpec((R, D_H), _rows),
        pl.BlockSpec((1, D_H), _full),
        pl.BlockSpec((R, DISW), _rows),
        pl.BlockSpec((D_H, D_H), _full),
    ],
    out_specs=pl.BlockSpec((R, D_H), _rows),
    out_shape=jax.ShapeDtypeStruct((N, D_H), _f32),
)


def _stage_b2_body(q_ref, xws_ref, b_ref, dis_ref, y_ref, wl_ref,
                   h_ref, xwsl_ref):
    q = q_ref[...]
    dis = dis_ref[...][:, 0:1]
    h_ref[...] = dis * (q[0] + q[1] - xws_ref[...]) + b_ref[...]
    xwsl_ref[...] = dis * jnp.dot(y_ref[...], wl_ref[...],
                                  preferred_element_type=_f32)


_stage_b2 = pl.pallas_call(
    _stage_b2_body,
    grid=(G,),
    in_specs=[
        pl.BlockSpec((NC, R, D_H), _rows3),
        pl.BlockSpec((R, D_H), _rows),
        pl.BlockSpec((1, D_H), _full),
        pl.BlockSpec((R, DISW), _rows),
        pl.BlockSpec((R, D_OUT), _rows),
        pl.BlockSpec((D_OUT, D_OUT), _full),
    ],
    out_specs=[pl.BlockSpec((R, D_H), _rows), pl.BlockSpec((R, D_OUT), _rows)],
    out_shape=[jax.ShapeDtypeStruct((N, D_H), _f32),
               jax.ShapeDtypeStruct((N, D_OUT), _f32)],
)


def _stage_b3_body(q_ref, xws_ref, b_ref, dis_ref, y_ref, m_ref, w_ref,
                   out_ref):
    q = q_ref[...]
    dis = dis_ref[...][:, 0:1]
    xl = dis * (q[0] + q[1] - xws_ref[...]) + b_ref[...]
    xl = jnp.maximum(xl, 0.0)
    m = m_ref[...][:, 0:1]
    xl = m * y_ref[...] + (1.0 - m) * xl
    out_ref[...] = dis * jnp.dot(xl, w_ref[...], preferred_element_type=_f32)


_stage_b3 = pl.pallas_call(
    _stage_b3_body,
    grid=(G,),
    in_specs=[
        pl.BlockSpec((NC, R, D_OUT), _rows3),
        pl.BlockSpec((R, D_OUT), _rows),
        pl.BlockSpec((1, D_OUT), _full),
        pl.BlockSpec((R, DISW), _rows),
        pl.BlockSpec((R, D_OUT), _rows),
        pl.BlockSpec((R, DISW), _rows),
        pl.BlockSpec((D_OUT, D_OUT), _full),
    ],
    out_specs=pl.BlockSpec((R, D_OUT), _rows),
    out_shape=jax.ShapeDtypeStruct((N, D_OUT), _f32),
)


def _stage_b4_body(q_ref, xws_ref, b_ref, dis_ref, y_ref, m_ref, h_ref,
                   dw_ref, wh_ref, wl_ref, wd_ref, bf_ref, out_ref):
    q = q_ref[...]
    dis = dis_ref[...][:, 0:1]
    xl = dis * (q[0] + q[1] - xws_ref[...]) + b_ref[...]
    m = m_ref[...][:, 0:1]
    xl = m * y_ref[...] + (1.0 - m) * xl
    z = jnp.dot(h_ref[...], wh_ref[...], preferred_element_type=_f32)
    z = z + jnp.dot(xl, wl_ref[...], preferred_element_type=_f32)
    z = z + jnp.dot(dw_ref[...], wd_ref[...], preferred_element_type=_f32)
    out_ref[...] = jax.nn.sigmoid(z + bf_ref[...])


_stage_b4 = pl.pallas_call(
    _stage_b4_body,
    grid=(G,),
    in_specs=[
        pl.BlockSpec((NC, R, D_OUT), _rows3),
        pl.BlockSpec((R, D_OUT), _rows),
        pl.BlockSpec((1, D_OUT), _full),
        pl.BlockSpec((R, DISW), _rows),
        pl.BlockSpec((R, D_OUT), _rows),
        pl.BlockSpec((R, DISW), _rows),
        pl.BlockSpec((R, D_H), _rows),
        pl.BlockSpec((R, DW), _rows),
        pl.BlockSpec((D_H, D_OUT), _full),
        pl.BlockSpec((D_OUT, D_OUT), _full),
        pl.BlockSpec((DW, D_OUT), _full),
        pl.BlockSpec((1, D_OUT), _full),
    ],
    out_specs=pl.BlockSpec((R, D_OUT), _rows),
    out_shape=jax.ShapeDtypeStruct((N, D_OUT), _f32),
)


def kernel(x, y, edge_index, deep_walk_emb, label_input_mask,
           W0, b0, W1, b1, Wl, bl, Wf, bf):
    row2 = edge_index[0].astype(jnp.int32).reshape(NW, EPW)
    col2 = edge_index[1].astype(jnp.int32).reshape(NW, EPW)
    row = row2[:, :NCHUNK * K].reshape(NW, NCHUNK, K)
    col = col2[:, :NCHUNK * K].reshape(NW, NCHUNK, K)
    rowt = row2[:, NCHUNK * K:].reshape(NW, 1, KTAIL)
    colt = col2[:, NCHUNK * K:].reshape(NW, 1, KTAIL)
    ones16 = jnp.ones((N, DEGW), _f32)
    mf = jnp.broadcast_to(
        label_input_mask.astype(_f32)[:, None], (N, DISW))

    agg128 = _get_agg(D_H)
    agg64 = _get_agg(D_OUT)
    degp = _get_deg()(ones16, col, colt)
    dis, xws = _stage_a(x, W0, degp)
    q = agg128(xws, row, col, rowt, colt)
    xws = _stage_b1(q, xws, b0.reshape(1, D_H), dis, W1)
    q = agg128(xws, row, col, rowt, colt)
    h, xwsl = _stage_b2(q, xws, b1.reshape(1, D_H), dis, y, Wl[0])
    for j in range(NUM_LABEL_LAYERS - 1):
        q = agg64(xwsl, row, col, rowt, colt)
        xwsl = _stage_b3(q, xwsl, bl[j].reshape(1, D_OUT), dis, y, mf,
                         Wl[j + 1])
    q = agg64(xwsl, row, col, rowt, colt)
    out = _stage_b4(q, xwsl, bl[NUM_LABEL_LAYERS - 1].reshape(1, D_OUT),
                    dis, y, mf, h, deep_walk_emb,
                    Wf[:D_H], Wf[D_H:D_H + D_OUT], Wf[D_H + D_OUT:],
                    bf.reshape(1, D_OUT))
    return out


# speedup vs baseline: 1.0090x; 1.0090x over previous
import functools

import jax
import jax.numpy as jnp
from jax import lax
from jax.experimental import pallas as pl
from jax.experimental.pallas import tpu as pltpu
from jax.experimental.pallas import tpu_sc as plsc

N = 10000
E = 320000
D_IN = 128
D_H = 128
D_OUT = 64
DW = 64
NUM_LABEL_LAYERS = 10

NC = 2
NS = 16
NW = NC * NS
EPW = E // NW
K = 72
NCHUNK = EPW // K
KTAIL = EPW - NCHUNK * K
RPT = 624
RTAIL = N - NS * RPT
DEGW = 4

_f32 = jnp.float32


def _worker_id():
    return lax.axis_index("s") * NC + lax.axis_index("c")


def _make_agg(D):
    mesh = plsc.VectorSubcoreMesh(core_axis_name="c", subcore_axis_name="s", num_cores=NC, num_subcores=NS)

    @functools.partial(
        pl.kernel,
        out_type=jax.ShapeDtypeStruct((NC, N, D), _f32),
        mesh=mesh,
        compiler_params=pltpu.CompilerParams(use_tc_tiling_on_sc=False),
        scratch_types=[
            pltpu.VMEM((NCHUNK, K), jnp.int32),
            pltpu.VMEM((NCHUNK, K), jnp.int32),
            pltpu.VMEM((1, KTAIL), jnp.int32),
            pltpu.VMEM((1, KTAIL), jnp.int32),
            pltpu.VMEM((3, K, D), _f32),
            pltpu.VMEM_SHARED((N, D), _f32),
            [pltpu.SemaphoreType.DMA] * 3,
            [pltpu.SemaphoreType.DMA] * 3,
        ],
    )
    def agg(xws_hbm, row_hbm, col_hbm, rowt_hbm, colt_hbm, out_hbm,
            row_v, col_v, rowt_v, colt_v, buf_v, acc_s, gsem, tsem):
        c = lax.axis_index("c")
        s = lax.axis_index("s")
        wid = s * NC + c
        pltpu.sync_copy(row_hbm.at[wid], row_v)
        pltpu.sync_copy(col_hbm.at[wid], col_v)
        pltpu.sync_copy(rowt_hbm.at[wid], rowt_v)
        pltpu.sync_copy(colt_hbm.at[wid], colt_v)

        def gather(j, slot):
            return pltpu.make_async_copy(
                xws_hbm.at[row_v.at[j]], buf_v.at[slot], gsem[slot])

        def scatter(j, slot):
            return pltpu.make_async_copy(
                buf_v.at[slot], acc_s.at[col_v.at[j]], tsem[slot])

        gather(0, 0).start()
        gather(1, 1).start()
        base = pl.multiple_of(s * RPT, 8)
        pltpu.sync_copy(xws_hbm.at[pl.ds(base, RPT)],
                        acc_s.at[pl.ds(base, RPT)])

        @pl.when(s == 0)
        def _():
            pltpu.sync_copy(xws_hbm.at[pl.ds(NS * RPT, RTAIL)],
                            acc_s.at[pl.ds(NS * RPT, RTAIL)])

        plsc.subcore_barrier()

        def tri(q, carry):
            for k in range(3):
                j = 3 * q + k
                gather(j, k).wait()
                scatter(j, k).start(add=True)
                more = j + 2 < NCHUNK

                @pl.when(more & (j >= 1))
                def _():
                    scatter(j - 1, (k + 2) % 3).wait()
                    gather(j + 2, (k + 2) % 3).start()

                @pl.when(more & (j < 1))
                def _():
                    gather(j + 2, (k + 2) % 3).start()
            return carry

        lax.fori_loop(0, NCHUNK // 3, tri, 0)
        for j in range(3 * (NCHUNK // 3), NCHUNK):
            gather(j, j % 3).wait()
            scatter(j, j % 3).start(add=True)
        for j in range(max(0, NCHUNK - 3), NCHUNK):
            scatter(j, j % 3).wait()
        pltpu.sync_copy(xws_hbm.at[rowt_v.at[0]], buf_v.at[0, pl.ds(0, KTAIL)])
        pltpu.sync_copy(buf_v.at[0, pl.ds(0, KTAIL)],
                        acc_s.at[colt_v.at[0]], add=True)
        plsc.subcore_barrier()
        pltpu.sync_copy(acc_s.at[pl.ds(base, RPT)],
                        out_hbm.at[c, pl.ds(base, RPT)])

        @pl.when(s == 0)
        def _():
            pltpu.sync_copy(acc_s.at[pl.ds(NS * RPT, RTAIL)],
                            out_hbm.at[c, pl.ds(NS * RPT, RTAIL)])

    return agg


_get_agg = functools.lru_cache(maxsize=None)(_make_agg)


def _make_deg():
    mesh = plsc.VectorSubcoreMesh(core_axis_name="c", subcore_axis_name="s", num_cores=NC, num_subcores=NS)

    @functools.partial(
        pl.kernel,
        out_type=jax.ShapeDtypeStruct((NC, N, DEGW), _f32),
        mesh=mesh,
        compiler_params=pltpu.CompilerParams(use_tc_tiling_on_sc=False),
        scratch_types=[
            pltpu.VMEM((NCHUNK, K), jnp.int32),
            pltpu.VMEM((1, KTAIL), jnp.int32),
            pltpu.VMEM((K, DEGW), _f32),
            pltpu.VMEM_SHARED((N, DEGW), _f32),
            pltpu.SemaphoreType.DMA,
        ],
    )
    def deg(ones_hbm, col_hbm, colt_hbm, out_hbm, col_v, colt_v, ones_v,
            acc_s, sem):
        c = lax.axis_index("c")
        s = lax.axis_index("s")
        wid = s * NC + c
        pltpu.sync_copy(col_hbm.at[wid], col_v)
        pltpu.sync_copy(colt_hbm.at[wid], colt_v)
        pltpu.sync_copy(ones_hbm.at[pl.ds(0, K)], ones_v)
        base = pl.multiple_of(s * RPT, 8)
        pltpu.sync_copy(ones_hbm.at[pl.ds(base, RPT)],
                        acc_s.at[pl.ds(base, RPT)])

        @pl.when(s == 0)
        def _():
            pltpu.sync_copy(ones_hbm.at[pl.ds(NS * RPT, RTAIL)],
                            acc_s.at[pl.ds(NS * RPT, RTAIL)])

        plsc.subcore_barrier()

        def scat(j):
            return pltpu.make_async_copy(ones_v, acc_s.at[col_v.at[j]], sem)

        def start_body(j, carry):
            scat(j).start(add=True)
            return carry

        lax.fori_loop(0, NCHUNK, start_body, 0)
        tail = pltpu.make_async_copy(ones_v.at[pl.ds(0, KTAIL)],
                                     acc_s.at[colt_v.at[0]], sem)
        tail.start(add=True)

        def wait_body(j, carry):
            scat(0).wait()
            return carry

        lax.fori_loop(0, NCHUNK, wait_body, 0)
        tail.wait()
        plsc.subcore_barrier()
        pltpu.sync_copy(acc_s.at[pl.ds(base, RPT)],
                        out_hbm.at[c, pl.ds(base, RPT)])

        @pl.when(s == 0)
        def _():
            pltpu.sync_copy(acc_s.at[pl.ds(NS * RPT, RTAIL)],
                            out_hbm.at[c, pl.ds(NS * RPT, RTAIL)])

    return deg


_get_deg = functools.lru_cache(maxsize=None)(_make_deg)


R = 2000
G = N // R
DISW = 8


def _rows(i):
    return (i, 0)


def _rows3(i):
    return (0, i, 0)


def _full(i):
    return (0, 0)


def _stage_a_body(x_ref, w_ref, dp_ref, dis_ref, xws_ref):
    dp = dp_ref[...]
    deg = (dp[0] + dp[1])[:, 0:1] - 1.0
    dis = lax.rsqrt(deg)
    dis_ref[...] = jnp.broadcast_to(dis, (R, DISW))
    xws_ref[...] = dis * jnp.dot(x_ref[...], w_ref[...],
                                 preferred_element_type=_f32)


_stage_a = pl.pallas_call(
    _stage_a_body,
    grid=(G,),
    in_specs=[
        pl.BlockSpec((R, D_IN), _rows),
        pl.BlockSpec((D_IN, D_H), _full),
        pl.BlockSpec((NC, R, DEGW), _rows3),
    ],
    out_specs=[pl.BlockSpec((R, DISW), _rows), pl.BlockSpec((R, D_H), _rows)],
    out_shape=[jax.ShapeDtypeStruct((N, DISW), _f32),
               jax.ShapeDtypeStruct((N, D_H), _f32)],
)


def _stage_b1_body(q_ref, xws_ref, b_ref, dis_ref, w_ref, out_ref):
    q = q_ref[...]
    dis = dis_ref[...][:, 0:1]
    h = dis * (q[0] + q[1] - xws_ref[...]) + b_ref[...]
    h = jnp.maximum(h, 0.0)
    out_ref[...] = dis * jnp.dot(h, w_ref[...], preferred_element_type=_f32)


_stage_b1 = pl.pallas_call(
    _stage_b1_body,
    grid=(G,),
    in_specs=[
        pl.BlockSpec((NC, R, D_H), _rows3),
        pl.BlockSpec((R, D_H), _rows),
        pl.BlockSpec((1, D_H), _full),
        pl.BlockSpec((R, DISW), _rows),
        pl.BlockSpec((D_H, D_H), _full),
    ],
    out_specs=pl.BlockSpec((R, D_H), _rows),
    out_shape=jax.ShapeDtypeStruct((N, D_H), _f32),
)


def _stage_b2_body(q_ref, xws_ref, b_ref, dis_ref, y_ref, wl_ref,
                   h_ref, xwsl_ref):
    q = q_ref[...]
    dis = dis_ref[...][:, 0:1]
    h_ref[...] = dis * (q[0] + q[1] - xws_ref[...]) + b_ref[...]
    xwsl_ref[...] = dis * jnp.dot(y_ref[...], wl_ref[...],
                                  preferred_element_type=_f32)


_stage_b2 = pl.pallas_call(
    _stage_b2_body,
    grid=(G,),
    in_specs=[
        pl.BlockSpec((NC, R, D_H), _rows3),
        pl.BlockSpec((R, D_H), _rows),
        pl.BlockSpec((1, D_H), _full),
        pl.BlockSpec((R, DISW), _rows),
        pl.BlockSpec((R, D_OUT), _rows),
        pl.BlockSpec((D_OUT, D_OUT), _full),
    ],
    out_specs=[pl.BlockSpec((R, D_H), _rows), pl.BlockSpec((R, D_OUT), _rows)],
    out_shape=[jax.ShapeDtypeStruct((N, D_H), _f32),
               jax.ShapeDtypeStruct((N, D_OUT), _f32)],
)


def _stage_b3_body(q_ref, xws_ref, b_ref, dis_ref, y_ref, m_ref, w_ref,
                   out_ref):
    q = q_ref[...]
    dis = dis_ref[...][:, 0:1]
    xl = dis * (q[0] + q[1] - xws_ref[...]) + b_ref[...]
    xl = jnp.maximum(xl, 0.0)
    m = m_ref[...][:, 0:1]
    xl = m * y_ref[...] + (1.0 - m) * xl
    out_ref[...] = dis * jnp.dot(xl, w_ref[...], preferred_element_type=_f32)


_stage_b3 = pl.pallas_call(
    _stage_b3_body,
    grid=(G,),
    in_specs=[
        pl.BlockSpec((NC, R, D_OUT), _rows3),
        pl.BlockSpec((R, D_OUT), _rows),
        pl.BlockSpec((1, D_OUT), _full),
        pl.BlockSpec((R, DISW), _rows),
        pl.BlockSpec((R, D_OUT), _rows),
        pl.BlockSpec((R, DISW), _rows),
        pl.BlockSpec((D_OUT, D_OUT), _full),
    ],
    out_specs=pl.BlockSpec((R, D_OUT), _rows),
    out_shape=jax.ShapeDtypeStruct((N, D_OUT), _f32),
)


def _stage_b4_body(q_ref, xws_ref, b_ref, dis_ref, y_ref, m_ref, h_ref,
                   dw_ref, wh_ref, wl_ref, wd_ref, bf_ref, out_ref):
    q = q_ref[...]
    dis = dis_ref[...][:, 0:1]
    xl = dis * (q[0] + q[1] - xws_ref[...]) + b_ref[...]
    m = m_ref[...][:, 0:1]
    xl = m * y_ref[...] + (1.0 - m) * xl
    z = jnp.dot(h_ref[...], wh_ref[...], preferred_element_type=_f32)
    z = z + jnp.dot(xl, wl_ref[...], preferred_element_type=_f32)
    z = z + jnp.dot(dw_ref[...], wd_ref[...], preferred_element_type=_f32)
    out_ref[...] = jax.nn.sigmoid(z + bf_ref[...])


_stage_b4 = pl.pallas_call(
    _stage_b4_body,
    grid=(G,),
    in_specs=[
        pl.BlockSpec((NC, R, D_OUT), _rows3),
        pl.BlockSpec((R, D_OUT), _rows),
        pl.BlockSpec((1, D_OUT), _full),
        pl.BlockSpec((R, DISW), _rows),
        pl.BlockSpec((R, D_OUT), _rows),
        pl.BlockSpec((R, DISW), _rows),
        pl.BlockSpec((R, D_H), _rows),
        pl.BlockSpec((R, DW), _rows),
        pl.BlockSpec((D_H, D_OUT), _full),
        pl.BlockSpec((D_OUT, D_OUT), _full),
        pl.BlockSpec((DW, D_OUT), _full),
        pl.BlockSpec((1, D_OUT), _full),
    ],
    out_specs=pl.BlockSpec((R, D_OUT), _rows),
    out_shape=jax.ShapeDtypeStruct((N, D_OUT), _f32),
)


def kernel(x, y, edge_index, deep_walk_emb, label_input_mask,
           W0, b0, W1, b1, Wl, bl, Wf, bf):
    row2 = edge_index[0].astype(jnp.int32).reshape(NW, EPW)
    col2 = edge_index[1].astype(jnp.int32).reshape(NW, EPW)
    row = row2[:, :NCHUNK * K].reshape(NW, NCHUNK, K)
    col = col2[:, :NCHUNK * K].reshape(NW, NCHUNK, K)
    rowt = row2[:, NCHUNK * K:].reshape(NW, 1, KTAIL)
    colt = col2[:, NCHUNK * K:].reshape(NW, 1, KTAIL)
    ones16 = jnp.ones((N, DEGW), _f32)
    mf = jnp.broadcast_to(
        label_input_mask.astype(_f32)[:, None], (N, DISW))

    agg128 = _get_agg(D_H)
    agg64 = _get_agg(D_OUT)
    degp = _get_deg()(ones16, col, colt)
    dis, xws = _stage_a(x, W0, degp)
    q = agg128(xws, row, col, rowt, colt)
    xws = _stage_b1(q, xws, b0.reshape(1, D_H), dis, W1)
    q = agg128(xws, row, col, rowt, colt)
    h, xwsl = _stage_b2(q, xws, b1.reshape(1, D_H), dis, y, Wl[0])
    for j in range(NUM_LABEL_LAYERS - 1):
        q = agg64(xwsl, row, col, rowt, colt)
        xwsl = _stage_b3(q, xwsl, bl[j].reshape(1, D_OUT), dis, y, mf,
                         Wl[j + 1])
    q = agg64(xwsl, row, col, rowt, colt)
    out = _stage_b4(q, xwsl, bl[NUM_LABEL_LAYERS - 1].reshape(1, D_OUT),
                    dis, y, mf, h, deep_walk_emb,
                    Wf[:D_H], Wf[D_H:D_H + D_OUT], Wf[D_H + D_OUT:],
                    bf.reshape(1, D_OUT))
    return out


# speedup vs baseline: 1.0098x; 1.0008x over previous
import functools

import jax
import jax.numpy as jnp
from jax import lax
from jax.experimental import pallas as pl
from jax.experimental.pallas import tpu as pltpu
from jax.experimental.pallas import tpu_sc as plsc

N = 10000
E = 320000
D_IN = 128
D_H = 128
D_OUT = 64
DW = 64
NUM_LABEL_LAYERS = 10

NC = 2
NS = 16
NW = NC * NS
EPW = E // NW
K = 72
NCHUNK = EPW // K
KTAIL = EPW - NCHUNK * K
RPT = 624
RTAIL = N - NS * RPT
DEGW = 4

_f32 = jnp.float32


def _make_agg(D):
    mesh = plsc.VectorSubcoreMesh(core_axis_name="c", subcore_axis_name="s", num_cores=NC, num_subcores=NS)

    @functools.partial(
        pl.kernel,
        out_type=jax.ShapeDtypeStruct((NC, N, D), _f32),
        mesh=mesh,
        compiler_params=pltpu.CompilerParams(use_tc_tiling_on_sc=False),
        scratch_types=[
            pltpu.VMEM((NCHUNK, K), jnp.int32),
            pltpu.VMEM((NCHUNK, K), jnp.int32),
            pltpu.VMEM((1, KTAIL), jnp.int32),
            pltpu.VMEM((1, KTAIL), jnp.int32),
            pltpu.VMEM((3, K, D), _f32),
            pltpu.VMEM_SHARED((N, D), _f32),
            [pltpu.SemaphoreType.DMA] * 3,
            [pltpu.SemaphoreType.DMA] * 3,
        ],
    )
    def agg(xws_hbm, row_hbm, col_hbm, rowt_hbm, colt_hbm, out_hbm,
            row_v, col_v, rowt_v, colt_v, buf_v, acc_s, gsem, tsem):
        c = lax.axis_index("c")
        s = lax.axis_index("s")
        wid = s * NC + c
        pltpu.sync_copy(row_hbm.at[wid], row_v)
        pltpu.sync_copy(col_hbm.at[wid], col_v)
        pltpu.sync_copy(rowt_hbm.at[wid], rowt_v)
        pltpu.sync_copy(colt_hbm.at[wid], colt_v)

        def gather(j, slot):
            return pltpu.make_async_copy(
                xws_hbm.at[row_v.at[j]], buf_v.at[slot], gsem[slot])

        def scatter(j, slot):
            return pltpu.make_async_copy(
                buf_v.at[slot], acc_s.at[col_v.at[j]], tsem[slot])

        gather(0, 0).start()
        gather(1, 1).start()
        base = pl.multiple_of(s * RPT, 8)
        pltpu.sync_copy(xws_hbm.at[pl.ds(base, RPT)],
                        acc_s.at[pl.ds(base, RPT)])

        @pl.when(s == 0)
        def _():
            pltpu.sync_copy(xws_hbm.at[pl.ds(NS * RPT, RTAIL)],
                            acc_s.at[pl.ds(NS * RPT, RTAIL)])

        plsc.subcore_barrier()

        def tri(q, carry):
            for k in range(3):
                j = 3 * q + k
                gather(j, k).wait()
                scatter(j, k).start(add=True)
                more = j + 2 < NCHUNK

                @pl.when(more & (j >= 1))
                def _():
                    scatter(j - 1, (k + 2) % 3).wait()
                    gather(j + 2, (k + 2) % 3).start()

                @pl.when(more & (j < 1))
                def _():
                    gather(j + 2, (k + 2) % 3).start()
            return carry

        lax.fori_loop(0, NCHUNK // 3, tri, 0)
        for j in range(3 * (NCHUNK // 3), NCHUNK):
            gather(j, j % 3).wait()
            scatter(j, j % 3).start(add=True)
        for j in range(max(0, NCHUNK - 3), NCHUNK):
            scatter(j, j % 3).wait()
        pltpu.sync_copy(xws_hbm.at[rowt_v.at[0]], buf_v.at[0, pl.ds(0, KTAIL)])
        pltpu.sync_copy(buf_v.at[0, pl.ds(0, KTAIL)],
                        acc_s.at[colt_v.at[0]], add=True)
        plsc.subcore_barrier()
        pltpu.sync_copy(acc_s.at[pl.ds(base, RPT)],
                        out_hbm.at[c, pl.ds(base, RPT)])

        @pl.when(s == 0)
        def _():
            pltpu.sync_copy(acc_s.at[pl.ds(NS * RPT, RTAIL)],
                            out_hbm.at[c, pl.ds(NS * RPT, RTAIL)])

    return agg


_get_agg = functools.lru_cache(maxsize=None)(_make_agg)


def _make_deg():
    mesh = plsc.VectorSubcoreMesh(core_axis_name="c", subcore_axis_name="s", num_cores=NC, num_subcores=NS)

    @functools.partial(
        pl.kernel,
        out_type=jax.ShapeDtypeStruct((NC, N, DEGW), _f32),
        mesh=mesh,
        compiler_params=pltpu.CompilerParams(use_tc_tiling_on_sc=False),
        scratch_types=[
            pltpu.VMEM((NCHUNK, K), jnp.int32),
            pltpu.VMEM((1, KTAIL), jnp.int32),
            pltpu.VMEM((K, DEGW), _f32),
            pltpu.VMEM_SHARED((N, DEGW), _f32),
            pltpu.SemaphoreType.DMA,
        ],
    )
    def deg(ones_hbm, col_hbm, colt_hbm, out_hbm, col_v, colt_v, ones_v,
            acc_s, sem):
        c = lax.axis_index("c")
        s = lax.axis_index("s")
        wid = s * NC + c
        pltpu.sync_copy(col_hbm.at[wid], col_v)
        pltpu.sync_copy(colt_hbm.at[wid], colt_v)
        pltpu.sync_copy(ones_hbm.at[pl.ds(0, K)], ones_v)
        base = pl.multiple_of(s * RPT, 8)
        pltpu.sync_copy(ones_hbm.at[pl.ds(base, RPT)],
                        acc_s.at[pl.ds(base, RPT)])

        @pl.when(s == 0)
        def _():
            pltpu.sync_copy(ones_hbm.at[pl.ds(NS * RPT, RTAIL)],
                            acc_s.at[pl.ds(NS * RPT, RTAIL)])

        plsc.subcore_barrier()

        def scat(j):
            return pltpu.make_async_copy(ones_v, acc_s.at[col_v.at[j]], sem)

        def start_body(j, carry):
            scat(j).start(add=True)
            return carry

        lax.fori_loop(0, NCHUNK, start_body, 0)
        tail = pltpu.make_async_copy(ones_v.at[pl.ds(0, KTAIL)],
                                     acc_s.at[colt_v.at[0]], sem)
        tail.start(add=True)

        def wait_body(j, carry):
            scat(0).wait()
            return carry

        lax.fori_loop(0, NCHUNK, wait_body, 0)
        tail.wait()
        plsc.subcore_barrier()
        pltpu.sync_copy(acc_s.at[pl.ds(base, RPT)],
                        out_hbm.at[c, pl.ds(base, RPT)])

        @pl.when(s == 0)
        def _():
            pltpu.sync_copy(acc_s.at[pl.ds(NS * RPT, RTAIL)],
                            out_hbm.at[c, pl.ds(NS * RPT, RTAIL)])

    return deg


_get_deg = functools.lru_cache(maxsize=None)(_make_deg)


R = 2000
G = N // R
DISW = 8


def _rows(i):
    return (i, 0)


def _rows3(i):
    return (0, i, 0)


def _full(i):
    return (0, 0)


def _stage_a_body(x_ref, w_ref, dp_ref, dis_ref, xws_ref):
    dp = dp_ref[...]
    deg = (dp[0] + dp[1])[:, 0:1] - 1.0
    dis = lax.rsqrt(deg)
    dis_ref[...] = jnp.broadcast_to(dis, (R, DISW))
    xws_ref[...] = dis * jnp.dot(x_ref[...], w_ref[...],
                                 preferred_element_type=_f32)


_stage_a = pl.pallas_call(
    _stage_a_body,
    grid=(G,),
    in_specs=[
        pl.BlockSpec((R, D_IN), _rows),
        pl.BlockSpec((D_IN, D_H), _full),
        pl.BlockSpec((NC, R, DEGW), _rows3),
    ],
    out_specs=[pl.BlockSpec((R, DISW), _rows), pl.BlockSpec((R, D_H), _rows)],
    out_shape=[jax.ShapeDtypeStruct((N, DISW), _f32),
               jax.ShapeDtypeStruct((N, D_H), _f32)],
)


def _stage_b1_body(q_ref, xws_ref, b_ref, dis_ref, w_ref, out_ref):
    q = q_ref[...]
    dis = dis_ref[...][:, 0:1]
    h = dis * (q[0] + q[1] - xws_ref[...]) + b_ref[...]
    h = jnp.maximum(h, 0.0)
    out_ref[...] = dis * jnp.dot(h, w_ref[...], preferred_element_type=_f32)


_stage_b1 = pl.pallas_call(
    _stage_b1_body,
    grid=(G,),
    in_specs=[
        pl.BlockSpec((NC, R, D_H), _rows3),
        pl.BlockSpec((R, D_H), _rows),
        pl.BlockSpec((1, D_H), _full),
        pl.BlockSpec((R, DISW), _rows),
        pl.BlockSpec((D_H, D_H), _full),
    ],
    out_specs=pl.BlockSpec((R, D_H), _rows),
    out_shape=jax.ShapeDtypeStruct((N, D_H), _f32),
)


def _stage_b2_body(q_ref, xws_ref, b_ref, dis_ref, y_ref, wl_ref,
                   h_ref, xwsl_ref):
    q = q_ref[...]
    dis = dis_ref[...][:, 0:1]
    h_ref[...] = dis * (q[0] + q[1] - xws_ref[...]) + b_ref[...]
    xwsl_ref[...] = dis * jnp.dot(y_ref[...], wl_ref[...],
                                  preferred_element_type=_f32)


_stage_b2 = pl.pallas_call(
    _stage_b2_body,
    grid=(G,),
    in_specs=[
        pl.BlockSpec((NC, R, D_H), _rows3),
        pl.BlockSpec((R, D_H), _rows),
        pl.BlockSpec((1, D_H), _full),
        pl.BlockSpec((R, DISW), _rows),
        pl.BlockSpec((R, D_OUT), _rows),
        pl.BlockSpec((D_OUT, D_OUT), _full),
    ],
    out_specs=[pl.BlockSpec((R, D_H), _rows), pl.BlockSpec((R, D_OUT), _rows)],
    out_shape=[jax.ShapeDtypeStruct((N, D_H), _f32),
               jax.ShapeDtypeStruct((N, D_OUT), _f32)],
)


def _stage_b3_body(q_ref, xws_ref, b_ref, dis_ref, y_ref, m_ref, w_ref,
                   out_ref):
    q = q_ref[...]
    dis = dis_ref[...][:, 0:1]
    xl = dis * (q[0] + q[1] - xws_ref[...]) + b_ref[...]
    xl = jnp.maximum(xl, 0.0)
    m = m_ref[...][:, 0:1]
    xl = m * y_ref[...] + (1.0 - m) * xl
    out_ref[...] = dis * jnp.dot(xl, w_ref[...], preferred_element_type=_f32)


_stage_b3 = pl.pallas_call(
    _stage_b3_body,
    grid=(G,),
    in_specs=[
        pl.BlockSpec((NC, R, D_OUT), _rows3),
        pl.BlockSpec((R, D_OUT), _rows),
        pl.BlockSpec((1, D_OUT), _full),
        pl.BlockSpec((R, DISW), _rows),
        pl.BlockSpec((R, D_OUT), _rows),
        pl.BlockSpec((R, DISW), _rows),
        pl.BlockSpec((D_OUT, D_OUT), _full),
    ],
    out_specs=pl.BlockSpec((R, D_OUT), _rows),
    out_shape=jax.ShapeDtypeStruct((N, D_OUT), _f32),
)


def _stage_b4_body(q_ref, xws_ref, b_ref, dis_ref, y_ref, m_ref, h_ref,
                   dw_ref, wh_ref, wl_ref, wd_ref, bf_ref, out_ref):
    q = q_ref[...]
    dis = dis_ref[...][:, 0:1]
    xl = dis * (q[0] + q[1] - xws_ref[...]) + b_ref[...]
    m = m_ref[...][:, 0:1]
    xl = m * y_ref[...] + (1.0 - m) * xl
    z = jnp.dot(h_ref[...], wh_ref[...], preferred_element_type=_f32)
    z = z + jnp.dot(xl, wl_ref[...], preferred_element_type=_f32)
    z = z + jnp.dot(dw_ref[...], wd_ref[...], preferred_element_type=_f32)
    out_ref[...] = jax.nn.sigmoid(z + bf_ref[...])


_stage_b4 = pl.pallas_call(
    _stage_b4_body,
    grid=(G,),
    in_specs=[
        pl.BlockSpec((NC, R, D_OUT), _rows3),
        pl.BlockSpec((R, D_OUT), _rows),
        pl.BlockSpec((1, D_OUT), _full),
        pl.BlockSpec((R, DISW), _rows),
        pl.BlockSpec((R, D_OUT), _rows),
        pl.BlockSpec((R, DISW), _rows),
        pl.BlockSpec((R, D_H), _rows),
        pl.BlockSpec((R, DW), _rows),
        pl.BlockSpec((D_H, D_OUT), _full),
        pl.BlockSpec((D_OUT, D_OUT), _full),
        pl.BlockSpec((DW, D_OUT), _full),
        pl.BlockSpec((1, D_OUT), _full),
    ],
    out_specs=pl.BlockSpec((R, D_OUT), _rows),
    out_shape=jax.ShapeDtypeStruct((N, D_OUT), _f32),
)


def kernel(x, y, edge_index, deep_walk_emb, label_input_mask,
           W0, b0, W1, b1, Wl, bl, Wf, bf):
    row2 = edge_index[0].astype(jnp.int32).reshape(NW, EPW)
    col2 = edge_index[1].astype(jnp.int32).reshape(NW, EPW)
    row = row2[:, :NCHUNK * K].reshape(NW, NCHUNK, K)
    col = col2[:, :NCHUNK * K].reshape(NW, NCHUNK, K)
    rowt = row2[:, NCHUNK * K:].reshape(NW, 1, KTAIL)
    colt = col2[:, NCHUNK * K:].reshape(NW, 1, KTAIL)
    ones16 = jnp.ones((N, DEGW), _f32)
    mf = jnp.broadcast_to(
        label_input_mask.astype(_f32)[:, None], (N, DISW))

    agg128 = _get_agg(D_H)
    agg64 = _get_agg(D_OUT)
    degp = _get_deg()(ones16, col, colt)
    dis, xws = _stage_a(x, W0, degp)
    q = agg128(xws, row, col, rowt, colt)
    xws = _stage_b1(q, xws, b0.reshape(1, D_H), dis, W1)
    q = agg128(xws, row, col, rowt, colt)
    h, xwsl = _stage_b2(q, xws, b1.reshape(1, D_H), dis, y, Wl[0])
    for j in range(NUM_LABEL_LAYERS - 1):
        q = agg64(xwsl, row, col, rowt, colt)
        xwsl = _stage_b3(q, xwsl, bl[j].reshape(1, D_OUT), dis, y, mf,
                         Wl[j + 1])
    q = agg64(xwsl, row, col, rowt, colt)
    out = _stage_b4(q, xwsl, bl[NUM_LABEL_LAYERS - 1].reshape(1, D_OUT),
                    dis, y, mf, h, deep_walk_emb,
                    Wf[:D_H], Wf[D_H:D_H + D_OUT], Wf[D_H + D_OUT:],
                    bf.reshape(1, D_OUT))
    return out
